# baseline (speedup 1.0000x reference)
import functools

import jax
import jax.numpy as jnp
from jax import lax
from jax.experimental import pallas as pl
from jax.experimental.pallas import tpu as pltpu
from jax.experimental.pallas import tpu_sc as plsc

BATCH = 32
H = 16
LORA = 512
ROPE = 64
HEAD = LORA + ROPE
KV = 2048
TOT = BATCH * KV
SPLITS = 8
PER = KV // SPLITS

_NC = 2
_NS = 16
_NW = _NC * _NS
_RPW = TOT // _NW
_CH = 64

_RBLK = 2048


def _repack_body(kl_in, kr_in, v_in, kl_ref, kr_ref, vl_ref):
    kl_ref[...] = kl_in[:, 0, :]
    iota = lax.broadcasted_iota(jnp.int32, (_RBLK, 128), 1)
    kr_ref[...] = jnp.where(iota < ROPE, kr_in[:, 0, :], 0.0)
    vl_ref[...] = v_in[:, 0, :]


_tc_repack = pl.pallas_call(
    _repack_body,
    grid=(TOT // _RBLK,),
    in_specs=[
        pl.BlockSpec((_RBLK, 1, LORA), lambda i: (i, 0, 0)),
        pl.BlockSpec((_RBLK, 1, 128), lambda i: (i, 0, LORA // 128)),
        pl.BlockSpec((_RBLK, 1, LORA), lambda i: (i, 0, 0)),
    ],
    out_specs=[
        pl.BlockSpec((_RBLK, LORA), lambda i: (i, 0)),
        pl.BlockSpec((_RBLK, 128), lambda i: (i, 0)),
        pl.BlockSpec((_RBLK, LORA), lambda i: (i, 0)),
    ],
    out_shape=[
        jax.ShapeDtypeStruct((TOT, LORA), jnp.float32),
        jax.ShapeDtypeStruct((TOT, 128), jnp.float32),
        jax.ShapeDtypeStruct((TOT, LORA), jnp.float32),
    ],
)


def _gather_body(klin, krope, vlin, idx_hbm, out_kl, out_kr, out_v,
                 idx_v, klb, krb, vb, sem_kl, sem_kr, sem_v):
    wid = lax.axis_index("s") * _NC + lax.axis_index("c")
    base = wid * _RPW
    pltpu.sync_copy(idx_hbm.at[pl.ds(base, _RPW)], idx_v)

    def chunk(c, carry):
        ixs = idx_v.at[pl.ds(c * _CH, _CH)]
        ckl = pltpu.async_copy(klin.at[ixs], klb, sem_kl)
        ckr = pltpu.async_copy(krope.at[ixs], krb, sem_kr)
        cv = pltpu.async_copy(vlin.at[ixs], vb, sem_v)
        ckl.wait()
        pltpu.sync_copy(klb, out_kl.at[pl.ds(base + c * _CH, _CH)])
        ckr.wait()
        pltpu.sync_copy(krb, out_kr.at[pl.ds(base + c * _CH, _CH)])
        cv.wait()
        pltpu.sync_copy(vb, out_v.at[pl.ds(base + c * _CH, _CH)])
        return carry

    lax.fori_loop(0, _RPW // _CH, chunk, 0)


@functools.cache
def _sc_gather():
    return functools.partial(
        pl.kernel,
        out_type=(
            jax.ShapeDtypeStruct((TOT, LORA), jnp.float32),
            jax.ShapeDtypeStruct((TOT, 128), jnp.float32),
            jax.ShapeDtypeStruct((TOT, LORA), jnp.float32),
        ),
        mesh=plsc.VectorSubcoreMesh(core_axis_name="c", subcore_axis_name="s"),
        scratch_types=[
            pltpu.VMEM((_RPW,), jnp.int32),
            pltpu.VMEM((_CH, LORA), jnp.float32),
            pltpu.VMEM((_CH, 128), jnp.float32),
            pltpu.VMEM((_CH, LORA), jnp.float32),
            pltpu.SemaphoreType.DMA,
            pltpu.SemaphoreType.DMA,
            pltpu.SemaphoreType.DMA,
        ],
    )(_gather_body)


def _flash_body(ql_ref, qr_ref, kl_ref, kr_ref, v_ref, o_ref):
    sm_scale = 1.0 / (HEAD ** 0.5)
    ql = ql_ref[0]
    qr = qr_ref[0]
    kl = kl_ref[...]
    kr = kr_ref[...]
    v = v_ref[...]
    qk = lax.dot_general(ql, kl, (((1,), (1,)), ((), ())),
                         preferred_element_type=jnp.float32)
    qk = qk + lax.dot_general(qr, kr, (((1,), (1,)), ((), ())),
                              preferred_element_type=jnp.float32)
    qk = qk * sm_scale
    m = jnp.max(qk, axis=1, keepdims=True)
    p = jnp.exp(qk - m)
    s = jnp.sum(p, axis=1, keepdims=True)
    acc = lax.dot_general(p, v, (((1,), (0,)), ((), ())),
                          preferred_element_type=jnp.float32)
    lse = jnp.broadcast_to(m + jnp.log(s), (H, 128))
    o_ref[0, 0] = jnp.concatenate([acc / s, lse], axis=1)


_tc_flash = pl.pallas_call(
    _flash_body,
    grid=(BATCH, SPLITS),
    in_specs=[
        pl.BlockSpec((1, H, LORA), lambda b, s: (b, 0, 0)),
        pl.BlockSpec((1, H, 128), lambda b, s: (b, 0, 0)),
        pl.BlockSpec((PER, LORA), lambda b, s: (b * SPLITS + s, 0)),
        pl.BlockSpec((PER, 128), lambda b, s: (b * SPLITS + s, 0)),
        pl.BlockSpec((PER, LORA), lambda b, s: (b * SPLITS + s, 0)),
    ],
    out_specs=pl.BlockSpec((1, 1, H, LORA + 128), lambda b, s: (b, s, 0, 0)),
    out_shape=jax.ShapeDtypeStruct((BATCH, SPLITS, H, LORA + 128),
                                   jnp.float32),
)


def kernel(q, k_buffer, v_buffer, kv_indptr, kv_indices, num_kv_splits,
           cos_sin_cache, positions, kv_lora_rank, rotary_dim):
    ql = q[:, :, :LORA]
    qr = jnp.pad(q[:, :, LORA:], ((0, 0), (0, 0), (0, 128 - ROPE)))
    klin, krope, vlin = _tc_repack(k_buffer, k_buffer, v_buffer)
    kxl, kxr, vx = _sc_gather()(klin, krope, vlin, kv_indices)
    out = _tc_flash(ql, qr, kxl, kxr, vx)
    att = out[..., :LORA + 1].transpose(0, 2, 1, 3)
    k_pe_tokens_out = jnp.zeros((1,), dtype=q.dtype)
    return (att, k_pe_tokens_out)


# speedup vs baseline: 1.0665x; 1.0665x over previous
import functools

import jax
import jax.numpy as jnp
from jax import lax
from jax.experimental import pallas as pl
from jax.experimental.pallas import tpu as pltpu
from jax.experimental.pallas import tpu_sc as plsc

BATCH = 32
H = 16
LORA = 512
ROPE = 64
HEAD = LORA + ROPE
KV = 2048
TOT = BATCH * KV
SPLITS = 8
PER = KV // SPLITS

_NC = 2
_NS = 16
_NW = _NC * _NS
_RPW = TOT // _NW
_CH = 64

_RBLK = 2048


def _repack_v_body(v_in, vl_ref):
    vl_ref[...] = v_in[:, 0, :]


_tc_repack_v = pl.pallas_call(
    _repack_v_body,
    grid=(TOT // _RBLK,),
    in_specs=[pl.BlockSpec((_RBLK, 1, LORA), lambda i: (i, 0, 0))],
    out_specs=pl.BlockSpec((_RBLK, LORA), lambda i: (i, 0)),
    out_shape=jax.ShapeDtypeStruct((TOT, LORA), jnp.float32),
)


def _repack_k_body(kl_in, kr_in, kl_ref, kr_ref):
    kl_ref[...] = kl_in[:, 0, :]
    iota = lax.broadcasted_iota(jnp.int32, (_RBLK, 128), 1)
    kr_ref[...] = jnp.where(iota < ROPE, kr_in[:, 0, :], 0.0)


_tc_repack_k = pl.pallas_call(
    _repack_k_body,
    grid=(TOT // _RBLK,),
    in_specs=[
        pl.BlockSpec((_RBLK, 1, LORA), lambda i: (i, 0, 0)),
        pl.BlockSpec((_RBLK, 1, 128), lambda i: (i, 0, LORA // 128)),
    ],
    out_specs=[
        pl.BlockSpec((_RBLK, LORA), lambda i: (i, 0)),
        pl.BlockSpec((_RBLK, 128), lambda i: (i, 0)),
    ],
    out_shape=[
        jax.ShapeDtypeStruct((TOT, LORA), jnp.float32),
        jax.ShapeDtypeStruct((TOT, 128), jnp.float32),
    ],
)


def _gather_v_body(vlin, idx_hbm, out_v, idx_v, vb0, vb1, sem):
    wid = lax.axis_index("s") * _NC + lax.axis_index("c")
    base = wid * _RPW
    pltpu.sync_copy(idx_hbm.at[pl.ds(base, _RPW)], idx_v)
    bufs = (vb0, vb1)

    def chunk(c, carry):
        for b in range(2):
            cv = pltpu.async_copy(
                vlin.at[idx_v.at[pl.ds((2 * c + b) * _CH, _CH)]], bufs[b], sem)
        for b in range(2):
            pltpu.make_async_copy(
                vlin.at[idx_v.at[pl.ds((2 * c + b) * _CH, _CH)]], bufs[b],
                sem).wait()
            pltpu.sync_copy(
                bufs[b], out_v.at[pl.ds(base + (2 * c + b) * _CH, _CH)])
        return carry

    lax.fori_loop(0, _RPW // (2 * _CH), chunk, 0)


@functools.cache
def _sc_gather_v():
    return functools.partial(
        pl.kernel,
        out_type=jax.ShapeDtypeStruct((TOT, LORA), jnp.float32),
        mesh=plsc.VectorSubcoreMesh(core_axis_name="c", subcore_axis_name="s"),
        scratch_types=[
            pltpu.VMEM((_RPW,), jnp.int32),
            pltpu.VMEM((_CH, LORA), jnp.float32),
            pltpu.VMEM((_CH, LORA), jnp.float32),
            pltpu.SemaphoreType.DMA,
        ],
    )(_gather_v_body)


def _gather_k_body(klin, krope, idx_hbm, out_kl, out_kr,
                   idx_v, klb0, klb1, krb0, krb1, sem_kl, sem_kr):
    wid = lax.axis_index("s") * _NC + lax.axis_index("c")
    base = wid * _RPW
    pltpu.sync_copy(idx_hbm.at[pl.ds(base, _RPW)], idx_v)
    klbufs = (klb0, klb1)
    krbufs = (krb0, krb1)

    def chunk(c, carry):
        for b in range(2):
            ix = idx_v.at[pl.ds((2 * c + b) * _CH, _CH)]
            pltpu.async_copy(klin.at[ix], klbufs[b], sem_kl)
            pltpu.async_copy(krope.at[ix], krbufs[b], sem_kr)
        for b in range(2):
            ix = idx_v.at[pl.ds((2 * c + b) * _CH, _CH)]
            pltpu.make_async_copy(klin.at[ix], klbufs[b], sem_kl).wait()
            pltpu.sync_copy(
                klbufs[b], out_kl.at[pl.ds(base + (2 * c + b) * _CH, _CH)])
            pltpu.make_async_copy(krope.at[ix], krbufs[b], sem_kr).wait()
            pltpu.sync_copy(
                krbufs[b], out_kr.at[pl.ds(base + (2 * c + b) * _CH, _CH)])
        return carry

    lax.fori_loop(0, _RPW // (2 * _CH), chunk, 0)


@functools.cache
def _sc_gather_k():
    return functools.partial(
        pl.kernel,
        out_type=(
            jax.ShapeDtypeStruct((TOT, LORA), jnp.float32),
            jax.ShapeDtypeStruct((TOT, 128), jnp.float32),
        ),
        mesh=plsc.VectorSubcoreMesh(core_axis_name="c", subcore_axis_name="s"),
        scratch_types=[
            pltpu.VMEM((_RPW,), jnp.int32),
            pltpu.VMEM((_CH, LORA), jnp.float32),
            pltpu.VMEM((_CH, LORA), jnp.float32),
            pltpu.VMEM((_CH, 128), jnp.float32),
            pltpu.VMEM((_CH, 128), jnp.float32),
            pltpu.SemaphoreType.DMA,
            pltpu.SemaphoreType.DMA,
        ],
    )(_gather_k_body)


def _flash_body(ql_ref, qr_ref, kl_ref, kr_ref, v_ref, o_ref):
    sm_scale = 1.0 / (HEAD ** 0.5)
    ql = ql_ref[0]
    qr = qr_ref[0]
    for h in range(2):
        kl = kl_ref[h * PER:(h + 1) * PER]
        kr = kr_ref[h * PER:(h + 1) * PER]
        v = v_ref[h * PER:(h + 1) * PER]
        qk = lax.dot_general(ql, kl, (((1,), (1,)), ((), ())),
                             preferred_element_type=jnp.float32)
        qk = qk + lax.dot_general(qr, kr, (((1,), (1,)), ((), ())),
                                  preferred_element_type=jnp.float32)
        qk = qk * sm_scale
        m = jnp.max(qk, axis=1, keepdims=True)
        p = jnp.exp(qk - m)
        s = jnp.sum(p, axis=1, keepdims=True)
        acc = lax.dot_general(p, v, (((1,), (0,)), ((), ())),
                              preferred_element_type=jnp.float32)
        lse = jnp.broadcast_to(m + jnp.log(s), (H, 128))
        o_ref[0, h] = jnp.concatenate([acc / s, lse], axis=1)


_tc_flash = pl.pallas_call(
    _flash_body,
    grid=(BATCH, SPLITS // 2),
    in_specs=[
        pl.BlockSpec((1, H, LORA), lambda b, s: (b, 0, 0)),
        pl.BlockSpec((1, H, 128), lambda b, s: (b, 0, 0)),
        pl.BlockSpec((2 * PER, LORA), lambda b, s: (b * (SPLITS // 2) + s, 0)),
        pl.BlockSpec((2 * PER, 128), lambda b, s: (b * (SPLITS // 2) + s, 0)),
        pl.BlockSpec((2 * PER, LORA), lambda b, s: (b * (SPLITS // 2) + s, 0)),
    ],
    out_specs=pl.BlockSpec((1, 2, H, LORA + 128), lambda b, s: (b, s, 0, 0)),
    out_shape=jax.ShapeDtypeStruct((BATCH, SPLITS, H, LORA + 128),
                                   jnp.float32),
)


def kernel(q, k_buffer, v_buffer, kv_indptr, kv_indices, num_kv_splits,
           cos_sin_cache, positions, kv_lora_rank, rotary_dim):
    ql = q[:, :, :LORA]
    qr = jnp.pad(q[:, :, LORA:], ((0, 0), (0, 0), (0, 128 - ROPE)))
    vlin = _tc_repack_v(v_buffer)
    vx = _sc_gather_v()(vlin, kv_indices)
    klin, krope = _tc_repack_k(k_buffer, k_buffer)
    kxl, kxr = _sc_gather_k()(klin, krope, kv_indices)
    out = _tc_flash(ql, qr, kxl, kxr, vx)
    att = out[..., :LORA + 1].transpose(0, 2, 1, 3)
    k_pe_tokens_out = jnp.zeros((1,), dtype=q.dtype)
    return (att, k_pe_tokens_out)


# speedup vs baseline: 1.1513x; 1.0795x over previous
import functools

import jax
import jax.numpy as jnp
from jax import lax
from jax.experimental import pallas as pl
from jax.experimental.pallas import tpu as pltpu
from jax.experimental.pallas import tpu_sc as plsc

BATCH = 32
H = 16
LORA = 512
ROPE = 64
HEAD = LORA + ROPE
KV = 2048
TOT = BATCH * KV
SPLITS = 8
PER = KV // SPLITS

_NC = 2
_NS = 16
_NW = _NC * _NS
_RPW = TOT // _NW
_CH = 64

_RBLK = 2048


def _repack_v_body(v_in, vl_ref):
    vl_ref[...] = v_in[:, 0, :]


_tc_repack_v = pl.pallas_call(
    _repack_v_body,
    grid=(TOT // _RBLK,),
    in_specs=[pl.BlockSpec((_RBLK, 1, LORA), lambda i: (i, 0, 0))],
    out_specs=pl.BlockSpec((_RBLK, LORA), lambda i: (i, 0)),
    out_shape=jax.ShapeDtypeStruct((TOT, LORA), jnp.float32),
)


def _repack_k_body(kl_in, kr_in, kl_ref, kr_ref):
    kl_ref[...] = kl_in[:, 0, :]
    iota = lax.broadcasted_iota(jnp.int32, (_RBLK, 128), 1)
    kr_ref[...] = jnp.where(iota < ROPE, kr_in[:, 0, :], 0.0)


_tc_repack_k = pl.pallas_call(
    _repack_k_body,
    grid=(TOT // _RBLK,),
    in_specs=[
        pl.BlockSpec((_RBLK, 1, LORA), lambda i: (i, 0, 0)),
        pl.BlockSpec((_RBLK, 1, 128), lambda i: (i, 0, LORA // 128)),
    ],
    out_specs=[
        pl.BlockSpec((_RBLK, LORA), lambda i: (i, 0)),
        pl.BlockSpec((_RBLK, 128), lambda i: (i, 0)),
    ],
    out_shape=[
        jax.ShapeDtypeStruct((TOT, LORA), jnp.float32),
        jax.ShapeDtypeStruct((TOT, 128), jnp.float32),
    ],
)


def _gather_v_body(vlin, idx_hbm, out_v, idx_v, vb0, vb1, sem):
    wid = lax.axis_index("s") * _NC + lax.axis_index("c")
    base = wid * _RPW
    pltpu.sync_copy(idx_hbm.at[pl.ds(base, _RPW)], idx_v)
    bufs = (vb0, vb1)

    def chunk(c, carry):
        for b in range(2):
            cv = pltpu.async_copy(
                vlin.at[idx_v.at[pl.ds((2 * c + b) * _CH, _CH)]], bufs[b], sem)
        for b in range(2):
            pltpu.make_async_copy(
                vlin.at[idx_v.at[pl.ds((2 * c + b) * _CH, _CH)]], bufs[b],
                sem).wait()
            pltpu.sync_copy(
                bufs[b], out_v.at[pl.ds(base + (2 * c + b) * _CH, _CH)])
        return carry

    lax.fori_loop(0, _RPW // (2 * _CH), chunk, 0)


@functools.cache
def _sc_gather_v():
    return functools.partial(
        pl.kernel,
        out_type=jax.ShapeDtypeStruct((TOT, LORA), jnp.float32),
        mesh=plsc.VectorSubcoreMesh(core_axis_name="c", subcore_axis_name="s"),
        scratch_types=[
            pltpu.VMEM((_RPW,), jnp.int32),
            pltpu.VMEM((_CH, LORA), jnp.float32),
            pltpu.VMEM((_CH, LORA), jnp.float32),
            pltpu.SemaphoreType.DMA,
        ],
    )(_gather_v_body)


def _gather_k_body(klin, krope, idx_hbm, out_kl, out_kr,
                   idx_v, klb0, klb1, krb0, krb1, sem_kl, sem_kr):
    wid = lax.axis_index("s") * _NC + lax.axis_index("c")
    base = wid * _RPW
    pltpu.sync_copy(idx_hbm.at[pl.ds(base, _RPW)], idx_v)
    klbufs = (klb0, klb1)
    krbufs = (krb0, krb1)

    def chunk(c, carry):
        for b in range(2):
            ix = idx_v.at[pl.ds((2 * c + b) * _CH, _CH)]
            pltpu.async_copy(klin.at[ix], klbufs[b], sem_kl)
            pltpu.async_copy(krope.at[ix], krbufs[b], sem_kr)
        for b in range(2):
            ix = idx_v.at[pl.ds((2 * c + b) * _CH, _CH)]
            pltpu.make_async_copy(klin.at[ix], klbufs[b], sem_kl).wait()
            pltpu.sync_copy(
                klbufs[b], out_kl.at[pl.ds(base + (2 * c + b) * _CH, _CH)])
            pltpu.make_async_copy(krope.at[ix], krbufs[b], sem_kr).wait()
            pltpu.sync_copy(
                krbufs[b], out_kr.at[pl.ds(base + (2 * c + b) * _CH, _CH)])
        return carry

    lax.fori_loop(0, _RPW // (2 * _CH), chunk, 0)


@functools.cache
def _sc_gather_k():
    return functools.partial(
        pl.kernel,
        out_type=(
            jax.ShapeDtypeStruct((TOT, LORA), jnp.float32),
            jax.ShapeDtypeStruct((TOT, 128), jnp.float32),
        ),
        mesh=plsc.VectorSubcoreMesh(core_axis_name="c", subcore_axis_name="s"),
        scratch_types=[
            pltpu.VMEM((_RPW,), jnp.int32),
            pltpu.VMEM((_CH, LORA), jnp.float32),
            pltpu.VMEM((_CH, LORA), jnp.float32),
            pltpu.VMEM((_CH, 128), jnp.float32),
            pltpu.VMEM((_CH, 128), jnp.float32),
            pltpu.SemaphoreType.DMA,
            pltpu.SemaphoreType.DMA,
        ],
    )(_gather_k_body)


def _flash_body(ql_ref, qr_ref, kl_ref, kr_ref, v_ref, o_ref):
    sm_scale = 1.0 / (HEAD ** 0.5)
    ql = ql_ref[0]
    qr = qr_ref[0]
    for h in range(SPLITS):
        kl = kl_ref[h * PER:(h + 1) * PER]
        kr = kr_ref[h * PER:(h + 1) * PER]
        v = v_ref[h * PER:(h + 1) * PER]
        qk = lax.dot_general(ql, kl, (((1,), (1,)), ((), ())),
                             preferred_element_type=jnp.float32)
        qk = qk + lax.dot_general(qr, kr, (((1,), (1,)), ((), ())),
                                  preferred_element_type=jnp.float32)
        qk = qk * sm_scale
        m = jnp.max(qk, axis=1, keepdims=True)
        p = jnp.exp(qk - m)
        s = jnp.sum(p, axis=1, keepdims=True)
        acc = lax.dot_general(p, v, (((1,), (0,)), ((), ())),
                              preferred_element_type=jnp.float32)
        o_ref[0, :, h, :LORA] = acc / s
        o_ref[0, :, h, LORA:] = m + jnp.log(s)


_tc_flash = pl.pallas_call(
    _flash_body,
    grid=(BATCH,),
    in_specs=[
        pl.BlockSpec((1, H, LORA), lambda b: (b, 0, 0)),
        pl.BlockSpec((1, H, 128), lambda b: (b, 0, 0)),
        pl.BlockSpec((KV, LORA), lambda b: (b, 0)),
        pl.BlockSpec((KV, 128), lambda b: (b, 0)),
        pl.BlockSpec((KV, LORA), lambda b: (b, 0)),
    ],
    out_specs=pl.BlockSpec((1, H, SPLITS, LORA + 1), lambda b: (b, 0, 0, 0)),
    out_shape=jax.ShapeDtypeStruct((BATCH, H, SPLITS, LORA + 1), jnp.float32),
)


def kernel(q, k_buffer, v_buffer, kv_indptr, kv_indices, num_kv_splits,
           cos_sin_cache, positions, kv_lora_rank, rotary_dim):
    ql = q[:, :, :LORA]
    qr = jnp.pad(q[:, :, LORA:], ((0, 0), (0, 0), (0, 128 - ROPE)))
    vlin = _tc_repack_v(v_buffer)
    vx = _sc_gather_v()(vlin, kv_indices)
    klin, krope = _tc_repack_k(k_buffer, k_buffer)
    kxl, kxr = _sc_gather_k()(klin, krope, kv_indices)
    att = _tc_flash(ql, qr, kxl, kxr, vx)
    k_pe_tokens_out = jnp.zeros((1,), dtype=q.dtype)
    return (att, k_pe_tokens_out)


# speedup vs baseline: 1.3982x; 1.2144x over previous
import functools

import jax
import jax.numpy as jnp
from jax import lax
from jax.experimental import pallas as pl
from jax.experimental.pallas import tpu as pltpu
from jax.experimental.pallas import tpu_sc as plsc

BATCH = 32
H = 16
LORA = 512
ROPE = 64
HEAD = LORA + ROPE
KV = 2048
TOT = BATCH * KV
SPLITS = 8
PER = KV // SPLITS

_NC = 2
_NS = 16
_NW = _NC * _NS
_RPW = TOT // _NW
_CH = 64

_RBLK = 4096


def _rope_pack_body(kr_in, kr_ref):
    iota = lax.broadcasted_iota(jnp.int32, (_RBLK, 128), 1)
    kr_ref[...] = jnp.where(iota < ROPE, kr_in[:, 0, :], 0.0)


_tc_rope_pack = pl.pallas_call(
    _rope_pack_body,
    grid=(TOT // _RBLK,),
    in_specs=[pl.BlockSpec((_RBLK, 1, 128), lambda i: (i, 0, LORA // 128))],
    out_specs=pl.BlockSpec((_RBLK, 128), lambda i: (i, 0)),
    out_shape=jax.ShapeDtypeStruct((TOT, 128), jnp.float32),
)


def _gather_v_body(v3d, idx_hbm, out_v, idx_v, vb0, vb1, sem):
    wid = lax.axis_index("s") * _NC + lax.axis_index("c")
    base = wid * _RPW
    pltpu.sync_copy(idx_hbm.at[pl.ds(base, _RPW)], idx_v)
    bufs = (vb0, vb1)

    def chunk(c, carry):
        for b in range(2):
            pltpu.async_copy(
                v3d.at[idx_v.at[pl.ds((2 * c + b) * _CH, _CH)], pl.ds(0, 1)],
                bufs[b], sem)
        for b in range(2):
            pltpu.make_async_copy(
                v3d.at[idx_v.at[pl.ds((2 * c + b) * _CH, _CH)], pl.ds(0, 1)],
                bufs[b], sem).wait()
            pltpu.sync_copy(
                bufs[b], out_v.at[pl.ds(base + (2 * c + b) * _CH, _CH)])
        return carry

    lax.fori_loop(0, _RPW // (2 * _CH), chunk, 0)


@functools.cache
def _sc_gather_v():
    return functools.partial(
        pl.kernel,
        out_type=jax.ShapeDtypeStruct((TOT, 1, LORA), jnp.float32),
        mesh=plsc.VectorSubcoreMesh(core_axis_name="c", subcore_axis_name="s"),
        scratch_types=[
            pltpu.VMEM((_RPW,), jnp.int32),
            pltpu.VMEM((_CH, 1, LORA), jnp.float32),
            pltpu.VMEM((_CH, 1, LORA), jnp.float32),
            pltpu.SemaphoreType.DMA,
        ],
    )(_gather_v_body)


def _gather_k_body(k3d, krope, idx_hbm, out_kl, out_kr,
                   idx_v, klb0, klb1, krb0, krb1, sem_kl, sem_kr):
    wid = lax.axis_index("s") * _NC + lax.axis_index("c")
    base = wid * _RPW
    pltpu.sync_copy(idx_hbm.at[pl.ds(base, _RPW)], idx_v)
    klbufs = (klb0, klb1)
    krbufs = (krb0, krb1)

    def chunk(c, carry):
        for b in range(2):
            ix = idx_v.at[pl.ds((2 * c + b) * _CH, _CH)]
            pltpu.async_copy(k3d.at[ix, pl.ds(0, 1), pl.ds(0, LORA)],
                             klbufs[b], sem_kl)
            pltpu.async_copy(krope.at[ix], krbufs[b], sem_kr)
        for b in range(2):
            ix = idx_v.at[pl.ds((2 * c + b) * _CH, _CH)]
            pltpu.make_async_copy(k3d.at[ix, pl.ds(0, 1), pl.ds(0, LORA)],
                                  klbufs[b], sem_kl).wait()
            pltpu.sync_copy(
                klbufs[b], out_kl.at[pl.ds(base + (2 * c + b) * _CH, _CH)])
            pltpu.make_async_copy(krope.at[ix], krbufs[b], sem_kr).wait()
            pltpu.sync_copy(
                krbufs[b], out_kr.at[pl.ds(base + (2 * c + b) * _CH, _CH)])
        return carry

    lax.fori_loop(0, _RPW // (2 * _CH), chunk, 0)


@functools.cache
def _sc_gather_k():
    return functools.partial(
        pl.kernel,
        out_type=(
            jax.ShapeDtypeStruct((TOT, 1, LORA), jnp.float32),
            jax.ShapeDtypeStruct((TOT, 128), jnp.float32),
        ),
        mesh=plsc.VectorSubcoreMesh(core_axis_name="c", subcore_axis_name="s"),
        scratch_types=[
            pltpu.VMEM((_RPW,), jnp.int32),
            pltpu.VMEM((_CH, 1, LORA), jnp.float32),
            pltpu.VMEM((_CH, 1, LORA), jnp.float32),
            pltpu.VMEM((_CH, 128), jnp.float32),
            pltpu.VMEM((_CH, 128), jnp.float32),
            pltpu.SemaphoreType.DMA,
            pltpu.SemaphoreType.DMA,
        ],
    )(_gather_k_body)


def _flash_body(ql_ref, qr_ref, kl_ref, kr_ref, v_ref, o_ref):
    sm_scale = 1.0 / (HEAD ** 0.5)
    ql = ql_ref[0]
    qr = qr_ref[0]
    for h in range(SPLITS):
        kl = kl_ref[h * PER:(h + 1) * PER, 0]
        kr = kr_ref[h * PER:(h + 1) * PER]
        v = v_ref[h * PER:(h + 1) * PER, 0]
        qk = lax.dot_general(ql, kl, (((1,), (1,)), ((), ())),
                             preferred_element_type=jnp.float32)
        qk = qk + lax.dot_general(qr, kr, (((1,), (1,)), ((), ())),
                                  preferred_element_type=jnp.float32)
        qk = qk * sm_scale
        m = jnp.max(qk, axis=1, keepdims=True)
        p = jnp.exp(qk - m)
        s = jnp.sum(p, axis=1, keepdims=True)
        acc = lax.dot_general(p, v, (((1,), (0,)), ((), ())),
                              preferred_element_type=jnp.float32)
        o_ref[0, :, h, :LORA] = acc / s
        o_ref[0, :, h, LORA:] = m + jnp.log(s)


_tc_flash = pl.pallas_call(
    _flash_body,
    grid=(BATCH,),
    in_specs=[
        pl.BlockSpec((1, H, LORA), lambda b: (b, 0, 0)),
        pl.BlockSpec((1, H, 128), lambda b: (b, 0, 0)),
        pl.BlockSpec((KV, 1, LORA), lambda b: (b, 0, 0)),
        pl.BlockSpec((KV, 128), lambda b: (b, 0)),
        pl.BlockSpec((KV, 1, LORA), lambda b: (b, 0, 0)),
    ],
    out_specs=pl.BlockSpec((1, H, SPLITS, LORA + 1), lambda b: (b, 0, 0, 0)),
    out_shape=jax.ShapeDtypeStruct((BATCH, H, SPLITS, LORA + 1), jnp.float32),
)


def kernel(q, k_buffer, v_buffer, kv_indptr, kv_indices, num_kv_splits,
           cos_sin_cache, positions, kv_lora_rank, rotary_dim):
    ql = q[:, :, :LORA]
    qr = jnp.pad(q[:, :, LORA:], ((0, 0), (0, 0), (0, 128 - ROPE)))
    vx = _sc_gather_v()(v_buffer, kv_indices)
    krope = _tc_rope_pack(k_buffer)
    kxl, kxr = _sc_gather_k()(k_buffer, krope, kv_indices)
    att = _tc_flash(ql, qr, kxl, kxr, vx)
    k_pe_tokens_out = jnp.zeros((1,), dtype=q.dtype)
    return (att, k_pe_tokens_out)


# speedup vs baseline: 1.3983x; 1.0001x over previous
import functools

import jax
import jax.numpy as jnp
from jax import lax
from jax.experimental import pallas as pl
from jax.experimental.pallas import tpu as pltpu
from jax.experimental.pallas import tpu_sc as plsc

BATCH = 32
H = 16
LORA = 512
ROPE = 64
HEAD = LORA + ROPE
KV = 2048
TOT = BATCH * KV
SPLITS = 8
PER = KV // SPLITS

_NC = 2
_NS = 16
_NW = _NC * _NS
_RPW = TOT // _NW
_CH = 64

_RBLK = 4096


def _rope_pack_body(kr_in, kr_ref):
    iota = lax.broadcasted_iota(jnp.int32, (_RBLK, 128), 1)
    kr_ref[...] = jnp.where(iota < ROPE, kr_in[:, 0, :], 0.0)


_tc_rope_pack = pl.pallas_call(
    _rope_pack_body,
    grid=(TOT // _RBLK,),
    in_specs=[pl.BlockSpec((_RBLK, 1, 128), lambda i: (i, 0, LORA // 128))],
    out_specs=pl.BlockSpec((_RBLK, 128), lambda i: (i, 0)),
    out_shape=jax.ShapeDtypeStruct((TOT, 128), jnp.float32),
)


def _gather_v_body(v3d, idx_hbm, out_v, idx_v, vb0, vb1, sem):
    wid = lax.axis_index("s") * _NC + lax.axis_index("c")
    base = wid * _RPW
    pltpu.sync_copy(idx_hbm.at[pl.ds(base, _RPW)], idx_v)
    bufs = (vb0, vb1)

    def chunk(c, carry):
        for b in range(2):
            pltpu.async_copy(
                v3d.at[idx_v.at[pl.ds((2 * c + b) * _CH, _CH)], pl.ds(0, 1)],
                bufs[b], sem)
        for b in range(2):
            pltpu.make_async_copy(
                v3d.at[idx_v.at[pl.ds((2 * c + b) * _CH, _CH)], pl.ds(0, 1)],
                bufs[b], sem).wait()
            pltpu.sync_copy(
                bufs[b], out_v.at[pl.ds(base + (2 * c + b) * _CH, _CH)])
        return carry

    lax.fori_loop(0, _RPW // (2 * _CH), chunk, 0)


@functools.cache
def _sc_gather_v():
    return functools.partial(
        pl.kernel,
        out_type=jax.ShapeDtypeStruct((TOT, 1, LORA), jnp.float32),
        mesh=plsc.VectorSubcoreMesh(core_axis_name="c", subcore_axis_name="s"),
        compiler_params=pltpu.CompilerParams(use_tc_tiling_on_sc=True),
        scratch_types=[
            pltpu.VMEM((_RPW,), jnp.int32),
            pltpu.VMEM((_CH, 1, LORA), jnp.float32),
            pltpu.VMEM((_CH, 1, LORA), jnp.float32),
            pltpu.SemaphoreType.DMA,
        ],
    )(_gather_v_body)


def _gather_k_body(k3d, krope, idx_hbm, out_kl, out_kr,
                   idx_v, klb0, klb1, krb0, krb1, sem_kl, sem_kr):
    wid = lax.axis_index("s") * _NC + lax.axis_index("c")
    base = wid * _RPW
    pltpu.sync_copy(idx_hbm.at[pl.ds(base, _RPW)], idx_v)
    klbufs = (klb0, klb1)
    krbufs = (krb0, krb1)

    def chunk(c, carry):
        for b in range(2):
            ix = idx_v.at[pl.ds((2 * c + b) * _CH, _CH)]
            pltpu.async_copy(k3d.at[ix, pl.ds(0, 1), pl.ds(0, LORA)],
                             klbufs[b], sem_kl)
            pltpu.async_copy(krope.at[ix], krbufs[b], sem_kr)
        for b in range(2):
            ix = idx_v.at[pl.ds((2 * c + b) * _CH, _CH)]
            pltpu.make_async_copy(k3d.at[ix, pl.ds(0, 1), pl.ds(0, LORA)],
                                  klbufs[b], sem_kl).wait()
            pltpu.sync_copy(
                klbufs[b], out_kl.at[pl.ds(base + (2 * c + b) * _CH, _CH)])
            pltpu.make_async_copy(krope.at[ix], krbufs[b], sem_kr).wait()
            pltpu.sync_copy(
                krbufs[b], out_kr.at[pl.ds(base + (2 * c + b) * _CH, _CH)])
        return carry

    lax.fori_loop(0, _RPW // (2 * _CH), chunk, 0)


@functools.cache
def _sc_gather_k():
    return functools.partial(
        pl.kernel,
        out_type=(
            jax.ShapeDtypeStruct((TOT, 1, LORA), jnp.float32),
            jax.ShapeDtypeStruct((TOT, 128), jnp.float32),
        ),
        mesh=plsc.VectorSubcoreMesh(core_axis_name="c", subcore_axis_name="s"),
        compiler_params=pltpu.CompilerParams(use_tc_tiling_on_sc=True),
        scratch_types=[
            pltpu.VMEM((_RPW,), jnp.int32),
            pltpu.VMEM((_CH, 1, LORA), jnp.float32),
            pltpu.VMEM((_CH, 1, LORA), jnp.float32),
            pltpu.VMEM((_CH, 128), jnp.float32),
            pltpu.VMEM((_CH, 128), jnp.float32),
            pltpu.SemaphoreType.DMA,
            pltpu.SemaphoreType.DMA,
        ],
    )(_gather_k_body)


def _flash_body(ql_ref, qr_ref, kl_ref, kr_ref, v_ref, o_ref):
    sm_scale = 1.0 / (HEAD ** 0.5)
    ql = ql_ref[0]
    qr = qr_ref[0]
    for h in range(SPLITS):
        kl = kl_ref[h * PER:(h + 1) * PER, 0]
        kr = kr_ref[h * PER:(h + 1) * PER]
        v = v_ref[h * PER:(h + 1) * PER, 0]
        qk = lax.dot_general(ql, kl, (((1,), (1,)), ((), ())),
                             preferred_element_type=jnp.float32)
        qk = qk + lax.dot_general(qr, kr, (((1,), (1,)), ((), ())),
                                  preferred_element_type=jnp.float32)
        qk = qk * sm_scale
        m = jnp.max(qk, axis=1, keepdims=True)
        p = jnp.exp(qk - m)
        s = jnp.sum(p, axis=1, keepdims=True)
        acc = lax.dot_general(p, v, (((1,), (0,)), ((), ())),
                              preferred_element_type=jnp.float32)
        o_ref[0, :, h, :LORA] = acc / s
        o_ref[0, :, h, LORA:] = m + jnp.log(s)


_tc_flash = pl.pallas_call(
    _flash_body,
    grid=(BATCH,),
    in_specs=[
        pl.BlockSpec((1, H, LORA), lambda b: (b, 0, 0)),
        pl.BlockSpec((1, H, 128), lambda b: (b, 0, 0)),
        pl.BlockSpec((KV, 1, LORA), lambda b: (b, 0, 0)),
        pl.BlockSpec((KV, 128), lambda b: (b, 0)),
        pl.BlockSpec((KV, 1, LORA), lambda b: (b, 0, 0)),
    ],
    out_specs=pl.BlockSpec((1, H, SPLITS, LORA + 1), lambda b: (b, 0, 0, 0)),
    out_shape=jax.ShapeDtypeStruct((BATCH, H, SPLITS, LORA + 1), jnp.float32),
)


def kernel(q, k_buffer, v_buffer, kv_indptr, kv_indices, num_kv_splits,
           cos_sin_cache, positions, kv_lora_rank, rotary_dim):
    ql = q[:, :, :LORA]
    qr = jnp.pad(q[:, :, LORA:], ((0, 0), (0, 0), (0, 128 - ROPE)))
    vx = _sc_gather_v()(v_buffer, kv_indices)
    krope = _tc_rope_pack(k_buffer)
    kxl, kxr = _sc_gather_k()(k_buffer, krope, kv_indices)
    att = _tc_flash(ql, qr, kxl, kxr, vx)
    k_pe_tokens_out = jnp.zeros((1,), dtype=q.dtype)
    return (att, k_pe_tokens_out)


# speedup vs baseline: 1.3988x; 1.0004x over previous
import functools

import jax
import jax.numpy as jnp
from jax import lax
from jax.experimental import pallas as pl
from jax.experimental.pallas import tpu as pltpu
from jax.experimental.pallas import tpu_sc as plsc

BATCH = 32
H = 16
LORA = 512
ROPE = 64
HEAD = LORA + ROPE
KV = 2048
TOT = BATCH * KV
SPLITS = 8
PER = KV // SPLITS

_NC = 2
_NS = 16
_NW = _NC * _NS
_RPW = TOT // _NW
_CH = 64

_RBLK = 4096


def _rope_pack_body(kr_in, kr_ref):
    iota = lax.broadcasted_iota(jnp.int32, (_RBLK, 128), 1)
    kr_ref[...] = jnp.where(iota < ROPE, kr_in[:, 0, :], 0.0)


_tc_rope_pack = pl.pallas_call(
    _rope_pack_body,
    grid=(TOT // _RBLK,),
    in_specs=[pl.BlockSpec((_RBLK, 1, 128), lambda i: (i, 0, LORA // 128))],
    out_specs=pl.BlockSpec((_RBLK, 128), lambda i: (i, 0)),
    out_shape=jax.ShapeDtypeStruct((TOT, 128), jnp.float32),
)


def _gather_v_body(v3d, idx_hbm, out_v, idx_v, vb0, vb1, sem):
    wid = lax.axis_index("s") * _NC + lax.axis_index("c")
    base = wid * _RPW
    pltpu.sync_copy(idx_hbm.at[pl.ds(base, _RPW)], idx_v)
    bufs = (vb0, vb1)

    def chunk(c, carry):
        for b in range(2):
            pltpu.async_copy(
                v3d.at[idx_v.at[pl.ds((2 * c + b) * _CH, _CH)], pl.ds(0, 1)],
                bufs[b], sem)
        for b in range(2):
            pltpu.make_async_copy(
                v3d.at[idx_v.at[pl.ds((2 * c + b) * _CH, _CH)], pl.ds(0, 1)],
                bufs[b], sem).wait()
            pltpu.sync_copy(
                bufs[b], out_v.at[pl.ds(base + (2 * c + b) * _CH, _CH)])
        return carry

    lax.fori_loop(0, _RPW // (2 * _CH), chunk, 0)


@functools.cache
def _sc_gather_v():
    return functools.partial(
        pl.kernel,
        out_type=jax.ShapeDtypeStruct((TOT, 1, LORA), jnp.float32),
        mesh=plsc.VectorSubcoreMesh(core_axis_name="c", subcore_axis_name="s"),
        compiler_params=pltpu.CompilerParams(use_tc_tiling_on_sc=True, needs_layout_passes=False),
        scratch_types=[
            pltpu.VMEM((_RPW,), jnp.int32),
            pltpu.VMEM((_CH, 1, LORA), jnp.float32),
            pltpu.VMEM((_CH, 1, LORA), jnp.float32),
            pltpu.SemaphoreType.DMA,
        ],
    )(_gather_v_body)


def _gather_k_body(k3d, krope, idx_hbm, out_kl, out_kr,
                   idx_v, klb0, klb1, krb0, krb1, sem_kl, sem_kr):
    wid = lax.axis_index("s") * _NC + lax.axis_index("c")
    base = wid * _RPW
    pltpu.sync_copy(idx_hbm.at[pl.ds(base, _RPW)], idx_v)
    klbufs = (klb0, klb1)
    krbufs = (krb0, krb1)

    def chunk(c, carry):
        for b in range(2):
            ix = idx_v.at[pl.ds((2 * c + b) * _CH, _CH)]
            pltpu.async_copy(k3d.at[ix, pl.ds(0, 1), pl.ds(0, LORA)],
                             klbufs[b], sem_kl)
            pltpu.async_copy(krope.at[ix], krbufs[b], sem_kr)
        for b in range(2):
            ix = idx_v.at[pl.ds((2 * c + b) * _CH, _CH)]
            pltpu.make_async_copy(k3d.at[ix, pl.ds(0, 1), pl.ds(0, LORA)],
                                  klbufs[b], sem_kl).wait()
            pltpu.sync_copy(
                klbufs[b], out_kl.at[pl.ds(base + (2 * c + b) * _CH, _CH)])
            pltpu.make_async_copy(krope.at[ix], krbufs[b], sem_kr).wait()
            pltpu.sync_copy(
                krbufs[b], out_kr.at[pl.ds(base + (2 * c + b) * _CH, _CH)])
        return carry

    lax.fori_loop(0, _RPW // (2 * _CH), chunk, 0)


@functools.cache
def _sc_gather_k():
    return functools.partial(
        pl.kernel,
        out_type=(
            jax.ShapeDtypeStruct((TOT, 1, LORA), jnp.float32),
            jax.ShapeDtypeStruct((TOT, 128), jnp.float32),
        ),
        mesh=plsc.VectorSubcoreMesh(core_axis_name="c", subcore_axis_name="s"),
        compiler_params=pltpu.CompilerParams(use_tc_tiling_on_sc=True, needs_layout_passes=False),
        scratch_types=[
            pltpu.VMEM((_RPW,), jnp.int32),
            pltpu.VMEM((_CH, 1, LORA), jnp.float32),
            pltpu.VMEM((_CH, 1, LORA), jnp.float32),
            pltpu.VMEM((_CH, 128), jnp.float32),
            pltpu.VMEM((_CH, 128), jnp.float32),
            pltpu.SemaphoreType.DMA,
            pltpu.SemaphoreType.DMA,
        ],
    )(_gather_k_body)


def _flash_body(ql_ref, qr_ref, kl_ref, kr_ref, v_ref, o_ref):
    sm_scale = 1.0 / (HEAD ** 0.5)
    ql = ql_ref[0]
    qr = qr_ref[0]
    for h in range(SPLITS):
        kl = kl_ref[h * PER:(h + 1) * PER, 0]
        kr = kr_ref[h * PER:(h + 1) * PER]
        v = v_ref[h * PER:(h + 1) * PER, 0]
        qk = lax.dot_general(ql, kl, (((1,), (1,)), ((), ())),
                             preferred_element_type=jnp.float32)
        qk = qk + lax.dot_general(qr, kr, (((1,), (1,)), ((), ())),
                                  preferred_element_type=jnp.float32)
        qk = qk * sm_scale
        m = jnp.max(qk, axis=1, keepdims=True)
        p = jnp.exp(qk - m)
        s = jnp.sum(p, axis=1, keepdims=True)
        acc = lax.dot_general(p, v, (((1,), (0,)), ((), ())),
                              preferred_element_type=jnp.float32)
        o_ref[0, :, h, :LORA] = acc / s
        o_ref[0, :, h, LORA:] = m + jnp.log(s)


_tc_flash = pl.pallas_call(
    _flash_body,
    grid=(BATCH,),
    in_specs=[
        pl.BlockSpec((1, H, LORA), lambda b: (b, 0, 0)),
        pl.BlockSpec((1, H, 128), lambda b: (b, 0, 0)),
        pl.BlockSpec((KV, 1, LORA), lambda b: (b, 0, 0)),
        pl.BlockSpec((KV, 128), lambda b: (b, 0)),
        pl.BlockSpec((KV, 1, LORA), lambda b: (b, 0, 0)),
    ],
    out_specs=pl.BlockSpec((1, H, SPLITS, LORA + 1), lambda b: (b, 0, 0, 0)),
    out_shape=jax.ShapeDtypeStruct((BATCH, H, SPLITS, LORA + 1), jnp.float32),
)


def kernel(q, k_buffer, v_buffer, kv_indptr, kv_indices, num_kv_splits,
           cos_sin_cache, positions, kv_lora_rank, rotary_dim):
    ql = q[:, :, :LORA]
    qr = jnp.pad(q[:, :, LORA:], ((0, 0), (0, 0), (0, 128 - ROPE)))
    vx = _sc_gather_v()(v_buffer, kv_indices)
    krope = _tc_rope_pack(k_buffer)
    kxl, kxr = _sc_gather_k()(k_buffer, krope, kv_indices)
    att = _tc_flash(ql, qr, kxl, kxr, vx)
    k_pe_tokens_out = jnp.zeros((1,), dtype=q.dtype)
    return (att, k_pe_tokens_out)


# speedup vs baseline: 1.4022x; 1.0024x over previous
import functools

import jax
import jax.numpy as jnp
from jax import lax
from jax.experimental import pallas as pl
from jax.experimental.pallas import tpu as pltpu
from jax.experimental.pallas import tpu_sc as plsc

BATCH = 32
H = 16
LORA = 512
ROPE = 64
HEAD = LORA + ROPE
KV = 2048
TOT = BATCH * KV
SPLITS = 8
PER = KV // SPLITS

_NC = 2
_NS = 16
_NW = _NC * _NS
_RPW = TOT // _NW
_CH = 32

_RBLK = 4096


def _rope_pack_body(kr_in, kr_ref):
    iota = lax.broadcasted_iota(jnp.int32, (_RBLK, 128), 1)
    kr_ref[...] = jnp.where(iota < ROPE, kr_in[:, 0, :], 0.0)


_tc_rope_pack = pl.pallas_call(
    _rope_pack_body,
    grid=(TOT // _RBLK,),
    in_specs=[pl.BlockSpec((_RBLK, 1, 128), lambda i: (i, 0, LORA // 128))],
    out_specs=pl.BlockSpec((_RBLK, 128), lambda i: (i, 0)),
    out_shape=jax.ShapeDtypeStruct((TOT, 128), jnp.float32),
)


def _gather_body(v3d, k3d, krope, idx_hbm, out_v, out_kl, out_kr,
                 idx_v, vb0, vb1, klb0, klb1, krb0, krb1,
                 sem_v, sem_kl, sem_kr):
    wid = lax.axis_index("s") * _NC + lax.axis_index("c")
    base = wid * _RPW
    pltpu.sync_copy(idx_hbm.at[pl.ds(base, _RPW)], idx_v)
    vbufs = (vb0, vb1)
    klbufs = (klb0, klb1)
    krbufs = (krb0, krb1)

    def chunk(c, carry):
        for b in range(2):
            ix = idx_v.at[pl.ds((2 * c + b) * _CH, _CH)]
            pltpu.async_copy(v3d.at[ix, pl.ds(0, 1)], vbufs[b], sem_v)
            pltpu.async_copy(k3d.at[ix, pl.ds(0, 1), pl.ds(0, LORA)],
                             klbufs[b], sem_kl)
            pltpu.async_copy(krope.at[ix], krbufs[b], sem_kr)
        for b in range(2):
            ix = idx_v.at[pl.ds((2 * c + b) * _CH, _CH)]
            row = pl.ds(base + (2 * c + b) * _CH, _CH)
            pltpu.make_async_copy(v3d.at[ix, pl.ds(0, 1)], vbufs[b],
                                  sem_v).wait()
            pltpu.sync_copy(vbufs[b], out_v.at[row])
            pltpu.make_async_copy(k3d.at[ix, pl.ds(0, 1), pl.ds(0, LORA)],
                                  klbufs[b], sem_kl).wait()
            pltpu.sync_copy(klbufs[b], out_kl.at[row])
            pltpu.make_async_copy(krope.at[ix], krbufs[b], sem_kr).wait()
            pltpu.sync_copy(krbufs[b], out_kr.at[row])
        return carry

    lax.fori_loop(0, _RPW // (2 * _CH), chunk, 0)


@functools.cache
def _sc_gather():
    return functools.partial(
        pl.kernel,
        out_type=(
            jax.ShapeDtypeStruct((TOT, 1, LORA), jnp.float32),
            jax.ShapeDtypeStruct((TOT, 1, LORA), jnp.float32),
            jax.ShapeDtypeStruct((TOT, 128), jnp.float32),
        ),
        mesh=plsc.VectorSubcoreMesh(core_axis_name="c", subcore_axis_name="s"),
        compiler_params=pltpu.CompilerParams(use_tc_tiling_on_sc=True),
        scratch_types=[
            pltpu.VMEM((_RPW,), jnp.int32),
            pltpu.VMEM((_CH, 1, LORA), jnp.float32),
            pltpu.VMEM((_CH, 1, LORA), jnp.float32),
            pltpu.VMEM((_CH, 1, LORA), jnp.float32),
            pltpu.VMEM((_CH, 1, LORA), jnp.float32),
            pltpu.VMEM((_CH, 128), jnp.float32),
            pltpu.VMEM((_CH, 128), jnp.float32),
            pltpu.SemaphoreType.DMA,
            pltpu.SemaphoreType.DMA,
            pltpu.SemaphoreType.DMA,
        ],
    )(_gather_body)


def _flash_body(ql_ref, qr_ref, kl_ref, kr_ref, v_ref, o_ref):
    sm_scale = 1.0 / (HEAD ** 0.5)
    ql = ql_ref[0]
    qr = qr_ref[0]
    for h in range(SPLITS):
        kl = kl_ref[h * PER:(h + 1) * PER, 0]
        kr = kr_ref[h * PER:(h + 1) * PER]
        v = v_ref[h * PER:(h + 1) * PER, 0]
        qk = lax.dot_general(ql, kl, (((1,), (1,)), ((), ())),
                             preferred_element_type=jnp.float32)
        qk = qk + lax.dot_general(qr, kr, (((1,), (1,)), ((), ())),
                                  preferred_element_type=jnp.float32)
        qk = qk * sm_scale
        m = jnp.max(qk, axis=1, keepdims=True)
        p = jnp.exp(qk - m)
        s = jnp.sum(p, axis=1, keepdims=True)
        acc = lax.dot_general(p, v, (((1,), (0,)), ((), ())),
                              preferred_element_type=jnp.float32)
        o_ref[0, :, h, :LORA] = acc / s
        o_ref[0, :, h, LORA:] = m + jnp.log(s)


_tc_flash = pl.pallas_call(
    _flash_body,
    grid=(BATCH,),
    in_specs=[
        pl.BlockSpec((1, H, LORA), lambda b: (b, 0, 0)),
        pl.BlockSpec((1, H, 128), lambda b: (b, 0, 0)),
        pl.BlockSpec((KV, 1, LORA), lambda b: (b, 0, 0)),
        pl.BlockSpec((KV, 128), lambda b: (b, 0)),
        pl.BlockSpec((KV, 1, LORA), lambda b: (b, 0, 0)),
    ],
    out_specs=pl.BlockSpec((1, H, SPLITS, LORA + 1), lambda b: (b, 0, 0, 0)),
    out_shape=jax.ShapeDtypeStruct((BATCH, H, SPLITS, LORA + 1), jnp.float32),
)


def kernel(q, k_buffer, v_buffer, kv_indptr, kv_indices, num_kv_splits,
           cos_sin_cache, positions, kv_lora_rank, rotary_dim):
    ql = q[:, :, :LORA]
    qr = jnp.pad(q[:, :, LORA:], ((0, 0), (0, 0), (0, 128 - ROPE)))
    krope = _tc_rope_pack(k_buffer)
    vx, kxl, kxr = _sc_gather()(v_buffer, k_buffer, krope, kv_indices)
    att = _tc_flash(ql, qr, kxl, kxr, vx)
    k_pe_tokens_out = jnp.zeros((1,), dtype=q.dtype)
    return (att, k_pe_tokens_out)
